# Initial kernel scaffold; baseline (speedup 1.0000x reference)
import jax
import jax.numpy as jnp
from jax.experimental import pallas as pl


def kernel(x, weights, indices, W1, W2, W3, Ws1, Ws2, Ws3):
    raise NotImplementedError("write your pallas kernel here")


# speedup vs baseline: 1.9335x; 1.9335x over previous
import functools

import jax
import jax.numpy as jnp
from jax import lax
from jax.experimental import pallas as pl
from jax.experimental.pallas import tpu as pltpu
from jax.experimental.pallas import tpu_sc as plsc

D = 2048
E = 16
I = 1024
S = 2048
K = 2
SI = 2 * I

BM = 256
RPT = S * K + E * BM
NB = RPT // BM

NRB = 2
NIC = 4
BMS = S // NRB
CSI = SI // NIC


def _route(indices, weights):
    flat_e = indices.reshape(-1).astype(jnp.int32)
    order = jnp.argsort(flat_e)
    sorted_e = flat_e[order]
    counts = jnp.bincount(flat_e, length=E)
    padded = ((counts + BM - 1) // BM) * BM
    ucum = jnp.cumsum(counts)
    pcum = jnp.cumsum(padded)
    i = jnp.arange(S * K)
    pos_sorted = (pcum[sorted_e] - padded[sorted_e]
                  + (i - (ucum[sorted_e] - counts[sorted_e]))).astype(jnp.int32)
    token_pad = jnp.zeros((RPT,), jnp.int32).at[pos_sorted].set(
        (order // K).astype(jnp.int32))
    w_pad = jnp.zeros((RPT,), jnp.float32).at[pos_sorted].set(
        weights.reshape(-1)[order])
    slot_pos = jnp.zeros((S * K,), jnp.int32).at[order].set(pos_sorted)
    pos = slot_pos.reshape(S, K).T
    bs = jnp.arange(NB, dtype=jnp.int32) * BM
    be = jnp.searchsorted(pcum, bs, side='right').astype(jnp.int32)
    active = bs < pcum[-1]
    be = jnp.where(active, be, sorted_e[-1])
    meta = jnp.stack([be, active.astype(jnp.int32)])
    return token_pad, w_pad, pos, meta


def _group_body(meta_ref, xs_ref, w1_ref, w3_ref, w2_ref, wr_ref, ys_ref):
    b = pl.program_id(0)

    @pl.when(meta_ref[1, b] == 1)
    def _():
        xb = xs_ref[...].astype(jnp.bfloat16)
        w1 = w1_ref[0].astype(jnp.bfloat16)
        w3 = w3_ref[0].astype(jnp.bfloat16)
        w2 = w2_ref[0].astype(jnp.bfloat16)
        a = lax.dot_general(xb, w1, (((1,), (1,)), ((), ())),
                            preferred_element_type=jnp.float32)
        c = lax.dot_general(xb, w3, (((1,), (1,)), ((), ())),
                            preferred_element_type=jnp.float32)
        h = (jax.nn.silu(a) * c).astype(jnp.bfloat16)
        out = lax.dot_general(h, w2, (((1,), (1,)), ((), ())),
                              preferred_element_type=jnp.float32)
        ys_ref[...] = out * wr_ref[0, 0, :][:, None]


def _grouped_mlp(meta, xs, W1, W3, W2, w_pad):
    w3d = w_pad.reshape(NB, 1, BM)
    grid_spec = pltpu.PrefetchScalarGridSpec(
        num_scalar_prefetch=1,
        grid=(NB,),
        in_specs=[
            pl.BlockSpec((BM, D), lambda b, m: (b, 0)),
            pl.BlockSpec((1, I, D), lambda b, m: (m[0, b], 0, 0)),
            pl.BlockSpec((1, I, D), lambda b, m: (m[0, b], 0, 0)),
            pl.BlockSpec((1, D, I), lambda b, m: (m[0, b], 0, 0)),
            pl.BlockSpec((1, 1, BM), lambda b, m: (b, 0, 0)),
        ],
        out_specs=pl.BlockSpec((BM, D), lambda b, m: (b, 0)),
    )
    return pl.pallas_call(
        _group_body,
        grid_spec=grid_spec,
        out_shape=jax.ShapeDtypeStruct((RPT, D), jnp.float32),
        compiler_params=pltpu.CompilerParams(
            dimension_semantics=("arbitrary",),
            vmem_limit_bytes=120 * 1024 * 1024,
        ),
    )(meta, xs, W1, W3, W2, w3d)


def _shared_body(x_ref, w1_ref, w3_ref, w2_ref, z_ref):
    ic = pl.program_id(1)
    xb = x_ref[...].astype(jnp.bfloat16)
    w1 = w1_ref[...].astype(jnp.bfloat16)
    w3 = w3_ref[...].astype(jnp.bfloat16)
    w2 = w2_ref[...].astype(jnp.bfloat16)
    a = lax.dot_general(xb, w1, (((1,), (1,)), ((), ())),
                        preferred_element_type=jnp.float32)
    c = lax.dot_general(xb, w3, (((1,), (1,)), ((), ())),
                        preferred_element_type=jnp.float32)
    h = (jax.nn.silu(a) * c).astype(jnp.bfloat16)
    out = lax.dot_general(h, w2, (((1,), (1,)), ((), ())),
                          preferred_element_type=jnp.float32)

    @pl.when(ic == 0)
    def _():
        z_ref[...] = out

    @pl.when(ic != 0)
    def _():
        z_ref[...] += out


def _shared_mlp(xf, Ws1, Ws3, Ws2):
    return pl.pallas_call(
        _shared_body,
        grid=(NRB, NIC),
        in_specs=[
            pl.BlockSpec((BMS, D), lambda r, ic: (r, 0)),
            pl.BlockSpec((CSI, D), lambda r, ic: (ic, 0)),
            pl.BlockSpec((CSI, D), lambda r, ic: (ic, 0)),
            pl.BlockSpec((D, CSI), lambda r, ic: (0, ic)),
        ],
        out_specs=pl.BlockSpec((BMS, D), lambda r, ic: (r, 0)),
        out_shape=jax.ShapeDtypeStruct((S, D), jnp.float32),
        compiler_params=pltpu.CompilerParams(
            dimension_semantics=("arbitrary", "arbitrary"),
            vmem_limit_bytes=120 * 1024 * 1024,
        ),
    )(xf, Ws1, Ws3, Ws2)


def _gather(xf, token_pad):
    return xf[token_pad]


def _combine(ys, pos, z):
    return ys[pos[0]] + ys[pos[1]] + z


def kernel(x, weights, indices, W1, W2, W3, Ws1, Ws2, Ws3):
    xf = x.reshape(-1, D)
    token_pad, w_pad, pos, meta = _route(indices, weights)
    xs = _gather(xf, token_pad)
    ys = _grouped_mlp(meta, xs, W1, W3, W2, w_pad)
    z = _shared_mlp(xf, Ws1, Ws3, Ws2)
    y = _combine(ys, pos, z)
    return y.reshape(x.shape)
